# baseline (speedup 1.0000x reference)
import jax
import jax.numpy as jnp
from jax.experimental import pallas as pl
from jax.experimental.pallas import tpu as pltpu

N = 16384
D = 256
H = 4
DH = 64
S = 8
B = 512
NB = N // B
KM_ITERS = 10
CH = 2048
NCH = N // CH

_INTERPRET = False


def _dotbf(a, b, dims):
    return jax.lax.dot_general(a.astype(jnp.bfloat16), b.astype(jnp.bfloat16),
                               (dims, ((), ())),
                               preferred_element_type=jnp.float32)


def _dothi(a, b, dims):
    return jax.lax.dot_general(a, b, (dims, ((), ())),
                               precision=jax.lax.Precision.HIGHEST,
                               preferred_element_type=jnp.float32)


def _elu1(x):
    return jnp.where(x > 0, x, jnp.exp(x) - 1.0) + 1.0


BA = 2048
NBA = N // BA


def _assign4_body(x_ref, x2_ref, caug_ref, oh_ref):
    x = x_ref[...]
    c4 = caug_ref[0:4, :]
    c2r = caug_ref[4:5, 0:4]
    xc = _dotbf(x, c4, ((1,), (1,)))
    dist = x2_ref[...][:, 0:4] - 2.0 * xc + c2r
    best = jnp.min(dist, axis=1, keepdims=True)
    eq = (dist == best).astype(jnp.float32)
    upper = (jax.lax.broadcasted_iota(jnp.int32, (4, 4), 0) <=
             jax.lax.broadcasted_iota(jnp.int32, (4, 4), 1)).astype(jnp.float32)
    cum = _dothi(eq, upper, ((1,), (0,)))
    oh4 = eq * (cum == 1.0).astype(jnp.float32)
    pad = (jax.lax.broadcasted_iota(jnp.int32, (4, S), 0) ==
           jax.lax.broadcasted_iota(jnp.int32, (4, S), 1)).astype(jnp.float32)
    oh_ref[...] = _dothi(oh4, pad, ((1,), (0,)))


def _assign4(full, x2b, caug):
    return pl.pallas_call(
        _assign4_body,
        grid=(NBA,),
        in_specs=[pl.BlockSpec((BA, D), lambda b: (b, 0)),
                  pl.BlockSpec((BA, S), lambda b: (b, 0)),
                  pl.BlockSpec((S, D), lambda b: (0, 0))],
        out_specs=pl.BlockSpec((BA, S), lambda b: (b, 0)),
        out_shape=jax.ShapeDtypeStruct((N, S), jnp.float32),
        interpret=_INTERPRET,
    )(full, x2b, caug)


def _assign8_body(x_ref, x2_ref, caug_ref, clj_ref):
    x = x_ref[...]
    c8 = caug_ref[0:8, :]
    c2r = caug_ref[8:9, 0:8]
    xc = _dotbf(x, c8, ((1,), (1,)))
    dist = x2_ref[...] - 2.0 * xc + c2r
    sr = jax.lax.broadcasted_iota(jnp.int32, (S, S), 0)
    sc = jax.lax.broadcasted_iota(jnp.int32, (S, S), 1)
    swap = ((sr ^ 1) == sc).astype(jnp.float32)
    dsw = _dothi(dist, swap, ((1,), (0,)))
    lane8 = jax.lax.broadcasted_iota(jnp.int32, (BA, S), 1)
    evenf = ((lane8 % 2) == 0).astype(jnp.float32)
    clj_ref[...] = (evenf * (dist <= dsw).astype(jnp.float32) +
                    (1.0 - evenf) * (dist < dsw).astype(jnp.float32))


def _assign8(d0, x2b, caug):
    return pl.pallas_call(
        _assign8_body,
        grid=(NBA,),
        in_specs=[pl.BlockSpec((BA, D), lambda b: (b, 0)),
                  pl.BlockSpec((BA, S), lambda b: (b, 0)),
                  pl.BlockSpec((S + 1, D), lambda b: (0, 0))],
        out_specs=pl.BlockSpec((BA, S), lambda b: (b, 0)),
        out_shape=jax.ShapeDtypeStruct((N, S), jnp.float32),
        interpret=_INTERPRET,
    )(d0, x2b, caug)


def _segkv_body(s_act, x_ref, oh_ref, cnts_ref, wk_ref, wv_ref, kv_ref,
                ks_ref, akv_ref, ak_ref):
    b = pl.program_id(0)

    @pl.when(b == 0)
    def _():
        akv_ref[...] = jnp.zeros_like(akv_ref)
        ak_ref[...] = jnp.zeros_like(ak_ref)

    x = x_ref[...]
    kk = _elu1(_dotbf(x, wk_ref[...], ((1,), (1,))))
    v = _dotbf(x, wv_ref[...], ((1,), (1,)))
    oh = oh_ref[...]
    cntsel = _dothi(oh, cnts_ref[:, 0:1], ((1,), (0,)))
    vd = v / jnp.maximum(cntsel, 1.0)
    for s in range(s_act):
        ksg = kk * oh[:, s:s + 1]
        akv_ref[s * D:(s + 1) * D, :] = akv_ref[s * D:(s + 1) * D, :] + _dotbf(
            ksg, vd, ((0,), (0,)))
        ak_ref[s:s + 1, :] = ak_ref[s:s + 1, :] + jnp.sum(
            ksg, axis=0, keepdims=True)

    @pl.when(b == NB - 1)
    def _():
        row = (jax.lax.broadcasted_iota(jnp.int32, (S * D, D), 0) % D) // DH
        col = jax.lax.broadcasted_iota(jnp.int32, (S * D, D), 1) // DH
        bd = (row == col).astype(jnp.float32)
        kv_ref[...] = akv_ref[...] * bd
        ks_ref[...] = ak_ref[...]


def _segkv(x, oh, cnts, wk, wv, s_act):
    body = lambda *refs: _segkv_body(s_act, *refs)
    return pl.pallas_call(
        body,
        grid=(NB,),
        in_specs=[pl.BlockSpec((B, D), lambda b: (b, 0)),
                  pl.BlockSpec((B, S), lambda b: (b, 0)),
                  pl.BlockSpec((S, 128), lambda b: (0, 0)),
                  pl.BlockSpec((D, D), lambda b: (0, 0)),
                  pl.BlockSpec((D, D), lambda b: (0, 0))],
        out_specs=[pl.BlockSpec((S * D, D), lambda b: (0, 0)),
                   pl.BlockSpec((S, D), lambda b: (0, 0))],
        out_shape=[jax.ShapeDtypeStruct((S * D, D), jnp.float32),
                   jax.ShapeDtypeStruct((S, D), jnp.float32)],
        scratch_shapes=[pltpu.VMEM((S * D, D), jnp.float32),
                        pltpu.VMEM((S, D), jnp.float32)],
        interpret=_INTERPRET,
    )(x, oh, cnts, wk, wv)


def _apply_body(s_act, final, *refs):
    if final:
        (x_ref, oh_ref, kv_ref, ks_ref, cnts_ref, wq_ref, wm_ref, m1_ref,
         m2_ref, ln_ref, orig_ref, gates_ref, y_ref) = refs
    else:
        (x_ref, oh_ref, kv_ref, ks_ref, cnts_ref, wq_ref, wm_ref, m1_ref,
         m2_ref, ln_ref, y_ref) = refs
    x = x_ref[...]
    oh = oh_ref[...]
    qq = _elu1(_dotbf(x, wq_ref[...], ((1,), (1,))))
    o = jnp.zeros((B, D), jnp.float32)
    for s in range(s_act):
        o = o + oh[:, s:s + 1] * _dotbf(qq, kv_ref[s * D:(s + 1) * D, :],
                                        ((1,), (0,)))
    ksel = _dothi(oh, ks_ref[...], ((1,), (0,)))
    ehr = (jax.lax.broadcasted_iota(jnp.int32, (D, H), 0) // DH ==
           jax.lax.broadcasted_iota(jnp.int32, (D, H), 1)).astype(jnp.float32)
    p = (qq.astype(jnp.bfloat16).astype(jnp.float32) *
         ksel.astype(jnp.bfloat16).astype(jnp.float32))
    den = _dothi(p, ehr, ((1,), (0,))) + 1e-6
    zf = _dothi(1.0 / den, ehr, ((1,), (1,)))
    cntsel = _dothi(oh, cnts_ref[:, 0:1], ((1,), (0,)))
    msg0 = (o * zf) * cntsel
    msg = _dotbf(msg0, wm_ref[...], ((1,), (1,)))
    mu = jnp.mean(msg, axis=1, keepdims=True)
    var = jnp.mean((msg - mu) ** 2, axis=1, keepdims=True)
    msg = (msg - mu) / jnp.sqrt(var + 1e-5) * ln_ref[0:1, :] + ln_ref[1:2, :]
    h = _dotbf(jnp.concatenate([x, msg], axis=1), m1_ref[...], ((1,), (1,)))
    h = jnp.maximum(h, 0.0)
    h2 = _dotbf(h, m2_ref[...], ((1,), (1,)))
    mu2 = jnp.mean(h2, axis=1, keepdims=True)
    var2 = jnp.mean((h2 - mu2) ** 2, axis=1, keepdims=True)
    msg2 = (h2 - mu2) / jnp.sqrt(var2 + 1e-5) * ln_ref[2:3, :] + ln_ref[3:4, :]
    y = x + msg2
    if final:
        g = _dothi(oh, gates_ref[:, 0:1], ((1,), (0,)))
        y = g * y + (1.0 - g) * orig_ref[...]
    y_ref[...] = y


def _apply(x, oh, kv, ks, cnts, wq, wm, m1, m2, ln, s_act,
           orig=None, gates=None):
    final = orig is not None
    body = lambda *refs: _apply_body(s_act, final, *refs)
    in_specs = [pl.BlockSpec((B, D), lambda b: (b, 0)),
                pl.BlockSpec((B, S), lambda b: (b, 0)),
                pl.BlockSpec((S * D, D), lambda b: (0, 0)),
                pl.BlockSpec((S, D), lambda b: (0, 0)),
                pl.BlockSpec((S, 128), lambda b: (0, 0)),
                pl.BlockSpec((D, D), lambda b: (0, 0)),
                pl.BlockSpec((D, D), lambda b: (0, 0)),
                pl.BlockSpec((2 * D, 2 * D), lambda b: (0, 0)),
                pl.BlockSpec((D, 2 * D), lambda b: (0, 0)),
                pl.BlockSpec((8, D), lambda b: (0, 0))]
    args = [x, oh, kv, ks, cnts, wq, wm, m1, m2, ln]
    if final:
        in_specs += [pl.BlockSpec((B, D), lambda b: (b, 0)),
                     pl.BlockSpec((S, 128), lambda b: (0, 0))]
        args += [orig, gates]
    return pl.pallas_call(
        body,
        grid=(NB,),
        in_specs=in_specs,
        out_specs=pl.BlockSpec((B, D), lambda b: (b, 0)),
        out_shape=jax.ShapeDtypeStruct((N, D), jnp.float32),
        interpret=_INTERPRET,
    )(*args)


def kernel(desc2d, desc3d,
           l1_q, l1_k, l1_v, l1_merge, l1_mlp1, l1_mlp2,
           l1_n1g, l1_n1b, l1_n2g, l1_n2b,
           l2_q, l2_k, l2_v, l2_merge, l2_mlp1, l2_mlp2,
           l2_n1g, l2_n1b, l2_n2g, l2_n2b):
    n2 = desc2d.shape[0]
    full = jnp.concatenate([desc2d, desc3d], axis=0)
    ln1 = jnp.concatenate([l1_n1g.reshape(1, D), l1_n1b.reshape(1, D),
                           l1_n2g.reshape(1, D), l1_n2b.reshape(1, D),
                           jnp.zeros((4, D), jnp.float32)], axis=0)
    ln2 = jnp.concatenate([l2_n1g.reshape(1, D), l2_n1b.reshape(1, D),
                           l2_n2g.reshape(1, D), l2_n2b.reshape(1, D),
                           jnp.zeros((4, D), jnp.float32)], axis=0)
    x2 = jnp.sum(full * full, axis=1, keepdims=True)
    x2b = jnp.broadcast_to(x2, (N, S))
    c = full[0:4, :]
    for _ in range(KM_ITERS):
        c2 = jnp.sum(c * c, axis=1)
        caug = jnp.zeros((S, D), jnp.float32).at[0:4, :].set(c)
        caug = caug.at[4, 0:4].set(c2)
        oh4 = _assign4(full, x2b, caug)
        cl = jnp.argmax(oh4[:, 0:4], axis=1).astype(jnp.int32)
        sums = jax.ops.segment_sum(full, cl, num_segments=4)
        counts = jnp.maximum(jnp.bincount(cl, length=4), 1).astype(full.dtype)
        c = sums / counts[:, None]
    cnt4 = jnp.bincount(cl, length=4)
    cnts4 = jnp.broadcast_to(
        jnp.pad(cnt4.astype(jnp.float32), (0, 4))[:, None], (S, 128))
    gates = jnp.broadcast_to(
        jnp.repeat((cnt4 > 20).astype(jnp.float32), 2)[:, None], (S, 128))

    kv1, ks1 = _segkv(full, oh4, cnts4, l1_k, l1_v, s_act=4)
    d0 = _apply(full, oh4, kv1, ks1, cnts4, l1_q, l1_merge, l1_mlp1, l1_mlp2,
                ln1, s_act=4)

    d0x2 = jnp.sum(d0 * d0, axis=1, keepdims=True)
    d0x2b = jnp.broadcast_to(d0x2, (N, S))
    ar = jnp.arange(N)
    crows = []
    for i in range(4):
        mask = (cl == i)
        order = jnp.argsort(jnp.where(mask, ar, N))
        crows.append(d0[order[0:2], :])
    c8 = jnp.concatenate(crows, axis=0)
    for _ in range(KM_ITERS):
        c2 = jnp.sum(c8 * c8, axis=1)
        caug = jnp.zeros((S + 1, D), jnp.float32).at[0:8, :].set(c8)
        caug = caug.at[8, 0:8].set(c2)
        clj = _assign8(d0, d0x2b, caug)
        newc = []
        for i in range(4):
            cl2 = clj[:, 2 * i + 1].astype(jnp.int32)
            maskf = (cl == i).astype(d0.dtype)
            sums = jax.ops.segment_sum(d0 * maskf[:, None], cl2,
                                       num_segments=2)
            counts = jnp.maximum(
                jax.ops.segment_sum((cl == i).astype(jnp.int32), cl2,
                                    num_segments=2), 1).astype(d0.dtype)
            newc.append(sums / counts[:, None])
        c8 = jnp.concatenate(newc, axis=0)
    oh4f = (cl[:, None] == jnp.arange(4)[None, :]).astype(jnp.float32)
    oh8 = jnp.repeat(oh4f, 2, axis=1) * clj
    jsel = jnp.take_along_axis(clj, (2 * cl + 1)[:, None],
                               axis=1)[:, 0].astype(jnp.int32)
    cnt8 = jnp.bincount(2 * cl + jsel, length=8)
    cnts8 = jnp.broadcast_to(cnt8.astype(jnp.float32)[:, None], (S, 128))

    kv2, ks2 = _segkv(d0, oh8, cnts8, l2_k, l2_v, s_act=8)
    out = _apply(d0, oh8, kv2, ks2, cnts8, l2_q, l2_merge, l2_mlp1, l2_mlp2,
                 ln2, s_act=8, orig=full, gates=gates)
    return out[:n2, :], out[n2:, :]


# speedup vs baseline: 1.8569x; 1.8569x over previous
import jax
import jax.numpy as jnp
from jax.experimental import pallas as pl
from jax.experimental.pallas import tpu as pltpu

N = 16384
D = 256
H = 4
DH = 64
S = 8
B = 512
NB = N // B
KM_ITERS = 10
CH = 2048
NCH = N // CH

_INTERPRET = False


def _dotbf(a, b, dims):
    return jax.lax.dot_general(a.astype(jnp.bfloat16), b.astype(jnp.bfloat16),
                               (dims, ((), ())),
                               preferred_element_type=jnp.float32)


def _dothi(a, b, dims):
    return jax.lax.dot_general(a, b, (dims, ((), ())),
                               precision=jax.lax.Precision.HIGHEST,
                               preferred_element_type=jnp.float32)


def _elu1(x):
    return jnp.where(x > 0, x, jnp.exp(x) - 1.0) + 1.0


BA = 2048
NBA = N // BA


def _assign4_body(x_ref, x2_ref, caug_ref, oh_ref):
    x = x_ref[...]
    c4 = caug_ref[0:4, :]
    c2r = caug_ref[4:5, 0:4]
    xc = _dotbf(x, c4, ((1,), (1,)))
    dist = x2_ref[...][:, 0:4] - 2.0 * xc + c2r
    best = jnp.min(dist, axis=1, keepdims=True)
    eq = (dist == best).astype(jnp.float32)
    upper = (jax.lax.broadcasted_iota(jnp.int32, (4, 4), 0) <=
             jax.lax.broadcasted_iota(jnp.int32, (4, 4), 1)).astype(jnp.float32)
    cum = _dothi(eq, upper, ((1,), (0,)))
    oh4 = eq * (cum == 1.0).astype(jnp.float32)
    pad = (jax.lax.broadcasted_iota(jnp.int32, (4, S), 0) ==
           jax.lax.broadcasted_iota(jnp.int32, (4, S), 1)).astype(jnp.float32)
    oh_ref[...] = _dothi(oh4, pad, ((1,), (0,)))


def _assign4(full, x2b, caug):
    return pl.pallas_call(
        _assign4_body,
        grid=(NBA,),
        in_specs=[pl.BlockSpec((BA, D), lambda b: (b, 0)),
                  pl.BlockSpec((BA, S), lambda b: (b, 0)),
                  pl.BlockSpec((S, D), lambda b: (0, 0))],
        out_specs=pl.BlockSpec((BA, S), lambda b: (b, 0)),
        out_shape=jax.ShapeDtypeStruct((N, S), jnp.float32),
        interpret=_INTERPRET,
    )(full, x2b, caug)


def _assign8_body(x_ref, x2_ref, caug_ref, clj_ref):
    x = x_ref[...]
    c8 = caug_ref[0:8, :]
    c2r = caug_ref[8:9, 0:8]
    xc = _dotbf(x, c8, ((1,), (1,)))
    dist = x2_ref[...] - 2.0 * xc + c2r
    sr = jax.lax.broadcasted_iota(jnp.int32, (S, S), 0)
    sc = jax.lax.broadcasted_iota(jnp.int32, (S, S), 1)
    swap = ((sr ^ 1) == sc).astype(jnp.float32)
    dsw = _dothi(dist, swap, ((1,), (0,)))
    lane8 = jax.lax.broadcasted_iota(jnp.int32, (BA, S), 1)
    evenf = ((lane8 % 2) == 0).astype(jnp.float32)
    clj_ref[...] = (evenf * (dist <= dsw).astype(jnp.float32) +
                    (1.0 - evenf) * (dist < dsw).astype(jnp.float32))


def _assign8(d0, x2b, caug):
    return pl.pallas_call(
        _assign8_body,
        grid=(NBA,),
        in_specs=[pl.BlockSpec((BA, D), lambda b: (b, 0)),
                  pl.BlockSpec((BA, S), lambda b: (b, 0)),
                  pl.BlockSpec((S + 1, D), lambda b: (0, 0))],
        out_specs=pl.BlockSpec((BA, S), lambda b: (b, 0)),
        out_shape=jax.ShapeDtypeStruct((N, S), jnp.float32),
        interpret=_INTERPRET,
    )(d0, x2b, caug)


def _segkv_body(s_act, x_ref, oh_ref, cnts_ref, wk_ref, wv_ref, kv_ref,
                ks_ref, akv_ref, ak_ref):
    b = pl.program_id(0)

    @pl.when(b == 0)
    def _():
        akv_ref[...] = jnp.zeros_like(akv_ref)
        ak_ref[...] = jnp.zeros_like(ak_ref)

    x = x_ref[...]
    kk = _elu1(_dotbf(x, wk_ref[...], ((1,), (1,))))
    v = _dotbf(x, wv_ref[...], ((1,), (1,)))
    oh = oh_ref[...]
    cntsel = _dothi(oh, cnts_ref[:, 0:1], ((1,), (0,)))
    vd = v / jnp.maximum(cntsel, 1.0)
    for s in range(s_act):
        ksg = kk * oh[:, s:s + 1]
        akv_ref[s * D:(s + 1) * D, :] = akv_ref[s * D:(s + 1) * D, :] + _dotbf(
            ksg, vd, ((0,), (0,)))
        ak_ref[s:s + 1, :] = ak_ref[s:s + 1, :] + jnp.sum(
            ksg, axis=0, keepdims=True)

    @pl.when(b == NB - 1)
    def _():
        row = (jax.lax.broadcasted_iota(jnp.int32, (S * D, D), 0) % D) // DH
        col = jax.lax.broadcasted_iota(jnp.int32, (S * D, D), 1) // DH
        bd = (row == col).astype(jnp.float32)
        kv_ref[...] = akv_ref[...] * bd
        ks_ref[...] = ak_ref[...]


def _segkv(x, oh, cnts, wk, wv, s_act):
    body = lambda *refs: _segkv_body(s_act, *refs)
    return pl.pallas_call(
        body,
        grid=(NB,),
        in_specs=[pl.BlockSpec((B, D), lambda b: (b, 0)),
                  pl.BlockSpec((B, S), lambda b: (b, 0)),
                  pl.BlockSpec((S, 128), lambda b: (0, 0)),
                  pl.BlockSpec((D, D), lambda b: (0, 0)),
                  pl.BlockSpec((D, D), lambda b: (0, 0))],
        out_specs=[pl.BlockSpec((S * D, D), lambda b: (0, 0)),
                   pl.BlockSpec((S, D), lambda b: (0, 0))],
        out_shape=[jax.ShapeDtypeStruct((S * D, D), jnp.float32),
                   jax.ShapeDtypeStruct((S, D), jnp.float32)],
        scratch_shapes=[pltpu.VMEM((S * D, D), jnp.float32),
                        pltpu.VMEM((S, D), jnp.float32)],
        interpret=_INTERPRET,
    )(x, oh, cnts, wk, wv)


def _apply_body(s_act, final, *refs):
    if final:
        (x_ref, oh_ref, kv_ref, ks_ref, cnts_ref, wq_ref, wm_ref, m1_ref,
         m2_ref, ln_ref, orig_ref, gates_ref, y_ref) = refs
    else:
        (x_ref, oh_ref, kv_ref, ks_ref, cnts_ref, wq_ref, wm_ref, m1_ref,
         m2_ref, ln_ref, y_ref) = refs
    x = x_ref[...]
    oh = oh_ref[...]
    qq = _elu1(_dotbf(x, wq_ref[...], ((1,), (1,))))
    o = jnp.zeros((B, D), jnp.float32)
    for s in range(s_act):
        o = o + oh[:, s:s + 1] * _dotbf(qq, kv_ref[s * D:(s + 1) * D, :],
                                        ((1,), (0,)))
    ksel = _dothi(oh, ks_ref[...], ((1,), (0,)))
    ehr = (jax.lax.broadcasted_iota(jnp.int32, (D, H), 0) // DH ==
           jax.lax.broadcasted_iota(jnp.int32, (D, H), 1)).astype(jnp.float32)
    p = (qq.astype(jnp.bfloat16).astype(jnp.float32) *
         ksel.astype(jnp.bfloat16).astype(jnp.float32))
    den = _dothi(p, ehr, ((1,), (0,))) + 1e-6
    zf = _dothi(1.0 / den, ehr, ((1,), (1,)))
    cntsel = _dothi(oh, cnts_ref[:, 0:1], ((1,), (0,)))
    msg0 = (o * zf) * cntsel
    msg = _dotbf(msg0, wm_ref[...], ((1,), (1,)))
    mu = jnp.mean(msg, axis=1, keepdims=True)
    var = jnp.mean((msg - mu) ** 2, axis=1, keepdims=True)
    msg = (msg - mu) / jnp.sqrt(var + 1e-5) * ln_ref[0:1, :] + ln_ref[1:2, :]
    h = _dotbf(jnp.concatenate([x, msg], axis=1), m1_ref[...], ((1,), (1,)))
    h = jnp.maximum(h, 0.0)
    h2 = _dotbf(h, m2_ref[...], ((1,), (1,)))
    mu2 = jnp.mean(h2, axis=1, keepdims=True)
    var2 = jnp.mean((h2 - mu2) ** 2, axis=1, keepdims=True)
    msg2 = (h2 - mu2) / jnp.sqrt(var2 + 1e-5) * ln_ref[2:3, :] + ln_ref[3:4, :]
    y = x + msg2
    if final:
        g = _dothi(oh, gates_ref[:, 0:1], ((1,), (0,)))
        y = g * y + (1.0 - g) * orig_ref[...]
    y_ref[...] = y


def _apply(x, oh, kv, ks, cnts, wq, wm, m1, m2, ln, s_act,
           orig=None, gates=None):
    final = orig is not None
    body = lambda *refs: _apply_body(s_act, final, *refs)
    in_specs = [pl.BlockSpec((B, D), lambda b: (b, 0)),
                pl.BlockSpec((B, S), lambda b: (b, 0)),
                pl.BlockSpec((S * D, D), lambda b: (0, 0)),
                pl.BlockSpec((S, D), lambda b: (0, 0)),
                pl.BlockSpec((S, 128), lambda b: (0, 0)),
                pl.BlockSpec((D, D), lambda b: (0, 0)),
                pl.BlockSpec((D, D), lambda b: (0, 0)),
                pl.BlockSpec((2 * D, 2 * D), lambda b: (0, 0)),
                pl.BlockSpec((D, 2 * D), lambda b: (0, 0)),
                pl.BlockSpec((8, D), lambda b: (0, 0))]
    args = [x, oh, kv, ks, cnts, wq, wm, m1, m2, ln]
    if final:
        in_specs += [pl.BlockSpec((B, D), lambda b: (b, 0)),
                     pl.BlockSpec((S, 128), lambda b: (0, 0))]
        args += [orig, gates]
    return pl.pallas_call(
        body,
        grid=(NB,),
        in_specs=in_specs,
        out_specs=pl.BlockSpec((B, D), lambda b: (b, 0)),
        out_shape=jax.ShapeDtypeStruct((N, D), jnp.float32),
        interpret=_INTERPRET,
    )(*args)


def kernel(desc2d, desc3d,
           l1_q, l1_k, l1_v, l1_merge, l1_mlp1, l1_mlp2,
           l1_n1g, l1_n1b, l1_n2g, l1_n2b,
           l2_q, l2_k, l2_v, l2_merge, l2_mlp1, l2_mlp2,
           l2_n1g, l2_n1b, l2_n2g, l2_n2b):
    n2 = desc2d.shape[0]
    full = jnp.concatenate([desc2d, desc3d], axis=0)
    ln1 = jnp.concatenate([l1_n1g.reshape(1, D), l1_n1b.reshape(1, D),
                           l1_n2g.reshape(1, D), l1_n2b.reshape(1, D),
                           jnp.zeros((4, D), jnp.float32)], axis=0)
    ln2 = jnp.concatenate([l2_n1g.reshape(1, D), l2_n1b.reshape(1, D),
                           l2_n2g.reshape(1, D), l2_n2b.reshape(1, D),
                           jnp.zeros((4, D), jnp.float32)], axis=0)
    x2 = jnp.sum(full * full, axis=1, keepdims=True)
    x2b = jnp.broadcast_to(x2, (N, S))
    c = full[0:4, :]
    for _ in range(KM_ITERS):
        c2 = jnp.sum(c * c, axis=1)
        caug = jnp.zeros((S, D), jnp.float32).at[0:4, :].set(c)
        caug = caug.at[4, 0:4].set(c2)
        oh4 = _assign4(full, x2b, caug)
        cl = jnp.argmax(oh4[:, 0:4], axis=1).astype(jnp.int32)
        sums = jax.ops.segment_sum(full, cl, num_segments=4)
        counts = jnp.maximum(jnp.bincount(cl, length=4), 1).astype(full.dtype)
        c = sums / counts[:, None]
    cnt4 = jnp.bincount(cl, length=4)
    cnts4 = jnp.broadcast_to(
        jnp.pad(cnt4.astype(jnp.float32), (0, 4))[:, None], (S, 128))
    gates = jnp.broadcast_to(
        jnp.repeat((cnt4 > 20).astype(jnp.float32), 2)[:, None], (S, 128))

    kv1, ks1 = _segkv(full, oh4, cnts4, l1_k, l1_v, s_act=4)
    d0 = _apply(full, oh4, kv1, ks1, cnts4, l1_q, l1_merge, l1_mlp1, l1_mlp2,
                ln1, s_act=4)

    d0x2 = jnp.sum(d0 * d0, axis=1, keepdims=True)
    d0x2b = jnp.broadcast_to(d0x2, (N, S))
    ar = jnp.arange(N)
    crows = []
    for i in range(4):
        mask = (cl == i)
        order = jnp.argsort(jnp.where(mask, ar, N))
        crows.append(d0[order[0:2], :])
    c8 = jnp.concatenate(crows, axis=0)
    for _ in range(KM_ITERS):
        c2 = jnp.sum(c8 * c8, axis=1)
        caug = jnp.zeros((S + 1, D), jnp.float32).at[0:8, :].set(c8)
        caug = caug.at[8, 0:8].set(c2)
        clj = _assign8(d0, d0x2b, caug)
        jown = jnp.take_along_axis(clj, (2 * cl + 1)[:, None],
                                   axis=1)[:, 0].astype(jnp.int32)
        seg = 2 * cl + jown
        sums = jax.ops.segment_sum(d0, seg, num_segments=8)
        counts = jnp.maximum(jnp.bincount(seg, length=8), 1).astype(d0.dtype)
        c8 = sums / counts[:, None]
    oh4f = (cl[:, None] == jnp.arange(4)[None, :]).astype(jnp.float32)
    oh8 = jnp.repeat(oh4f, 2, axis=1) * clj
    jsel = jnp.take_along_axis(clj, (2 * cl + 1)[:, None],
                               axis=1)[:, 0].astype(jnp.int32)
    cnt8 = jnp.bincount(2 * cl + jsel, length=8)
    cnts8 = jnp.broadcast_to(cnt8.astype(jnp.float32)[:, None], (S, 128))

    kv2, ks2 = _segkv(d0, oh8, cnts8, l2_k, l2_v, s_act=8)
    out = _apply(d0, oh8, kv2, ks2, cnts8, l2_q, l2_merge, l2_mlp1, l2_mlp2,
                 ln2, s_act=8, orig=full, gates=gates)
    return out[:n2, :], out[n2:, :]


# speedup vs baseline: 2.5180x; 1.3560x over previous
import jax
import jax.numpy as jnp
from jax.experimental import pallas as pl
from jax.experimental.pallas import tpu as pltpu

N = 16384
D = 256
H = 4
DH = 64
S = 8
B = 512
NB = N // B
KM_ITERS = 10
CH = 2048
NCH = N // CH

_INTERPRET = False


def _dotbf(a, b, dims):
    return jax.lax.dot_general(a.astype(jnp.bfloat16), b.astype(jnp.bfloat16),
                               (dims, ((), ())),
                               preferred_element_type=jnp.float32)


def _dothi(a, b, dims):
    return jax.lax.dot_general(a, b, (dims, ((), ())),
                               precision=jax.lax.Precision.HIGHEST,
                               preferred_element_type=jnp.float32)


def _elu1(x):
    return jnp.where(x > 0, x, jnp.exp(x) - 1.0) + 1.0


BA = 2048
NBA = N // BA


def _assign4_body(x_ref, x2_ref, caug_ref, oh_ref):
    x = x_ref[...]
    c4 = caug_ref[0:4, :]
    c2r = caug_ref[4:5, 0:4]
    xc = _dotbf(x, c4, ((1,), (1,)))
    dist = x2_ref[...][:, 0:4] - 2.0 * xc + c2r
    best = jnp.min(dist, axis=1, keepdims=True)
    eq = (dist == best).astype(jnp.float32)
    upper = (jax.lax.broadcasted_iota(jnp.int32, (4, 4), 0) <=
             jax.lax.broadcasted_iota(jnp.int32, (4, 4), 1)).astype(jnp.float32)
    cum = _dothi(eq, upper, ((1,), (0,)))
    oh4 = eq * (cum == 1.0).astype(jnp.float32)
    pad = (jax.lax.broadcasted_iota(jnp.int32, (4, S), 0) ==
           jax.lax.broadcasted_iota(jnp.int32, (4, S), 1)).astype(jnp.float32)
    oh_ref[...] = _dothi(oh4, pad, ((1,), (0,)))


def _assign4(full, x2b, caug):
    return pl.pallas_call(
        _assign4_body,
        grid=(NBA,),
        in_specs=[pl.BlockSpec((BA, D), lambda b: (b, 0)),
                  pl.BlockSpec((BA, S), lambda b: (b, 0)),
                  pl.BlockSpec((S, D), lambda b: (0, 0))],
        out_specs=pl.BlockSpec((BA, S), lambda b: (b, 0)),
        out_shape=jax.ShapeDtypeStruct((N, S), jnp.float32),
        interpret=_INTERPRET,
    )(full, x2b, caug)


def _assign8_body(x_ref, x2_ref, caug_ref, clj_ref):
    x = x_ref[...]
    c8 = caug_ref[0:8, :]
    c2r = caug_ref[8:9, 0:8]
    xc = _dotbf(x, c8, ((1,), (1,)))
    dist = x2_ref[...] - 2.0 * xc + c2r
    sr = jax.lax.broadcasted_iota(jnp.int32, (S, S), 0)
    sc = jax.lax.broadcasted_iota(jnp.int32, (S, S), 1)
    swap = ((sr ^ 1) == sc).astype(jnp.float32)
    dsw = _dothi(dist, swap, ((1,), (0,)))
    lane8 = jax.lax.broadcasted_iota(jnp.int32, (BA, S), 1)
    evenf = ((lane8 % 2) == 0).astype(jnp.float32)
    clj_ref[...] = (evenf * (dist <= dsw).astype(jnp.float32) +
                    (1.0 - evenf) * (dist < dsw).astype(jnp.float32))


def _assign8(d0, x2b, caug):
    return pl.pallas_call(
        _assign8_body,
        grid=(NBA,),
        in_specs=[pl.BlockSpec((BA, D), lambda b: (b, 0)),
                  pl.BlockSpec((BA, S), lambda b: (b, 0)),
                  pl.BlockSpec((S + 1, D), lambda b: (0, 0))],
        out_specs=pl.BlockSpec((BA, S), lambda b: (b, 0)),
        out_shape=jax.ShapeDtypeStruct((N, S), jnp.float32),
        interpret=_INTERPRET,
    )(d0, x2b, caug)


def _segkv_body(s_act, x_ref, oh_ref, cnts_ref, wk_ref, wv_ref, kv_ref,
                ks_ref, akv_ref, ak_ref):
    b = pl.program_id(0)

    @pl.when(b == 0)
    def _():
        akv_ref[...] = jnp.zeros_like(akv_ref)
        ak_ref[...] = jnp.zeros_like(ak_ref)

    x = x_ref[...]
    kk = _elu1(_dotbf(x, wk_ref[...], ((1,), (1,))))
    v = _dotbf(x, wv_ref[...], ((1,), (1,)))
    oh = oh_ref[...]
    cntsel = _dothi(oh, cnts_ref[:, 0:1], ((1,), (0,)))
    vd = v / jnp.maximum(cntsel, 1.0)
    for s in range(s_act):
        ksg = kk * oh[:, s:s + 1]
        akv_ref[s * D:(s + 1) * D, :] = akv_ref[s * D:(s + 1) * D, :] + _dotbf(
            ksg, vd, ((0,), (0,)))
        ak_ref[s:s + 1, :] = ak_ref[s:s + 1, :] + jnp.sum(
            ksg, axis=0, keepdims=True)

    @pl.when(b == NB - 1)
    def _():
        row = (jax.lax.broadcasted_iota(jnp.int32, (S * D, D), 0) % D) // DH
        col = jax.lax.broadcasted_iota(jnp.int32, (S * D, D), 1) // DH
        bd = (row == col).astype(jnp.float32)
        kv_ref[...] = akv_ref[...] * bd
        ks_ref[...] = ak_ref[...]


def _segkv(x, oh, cnts, wk, wv, s_act):
    body = lambda *refs: _segkv_body(s_act, *refs)
    return pl.pallas_call(
        body,
        grid=(NB,),
        in_specs=[pl.BlockSpec((B, D), lambda b: (b, 0)),
                  pl.BlockSpec((B, S), lambda b: (b, 0)),
                  pl.BlockSpec((S, 128), lambda b: (0, 0)),
                  pl.BlockSpec((D, D), lambda b: (0, 0)),
                  pl.BlockSpec((D, D), lambda b: (0, 0))],
        out_specs=[pl.BlockSpec((S * D, D), lambda b: (0, 0)),
                   pl.BlockSpec((S, D), lambda b: (0, 0))],
        out_shape=[jax.ShapeDtypeStruct((S * D, D), jnp.float32),
                   jax.ShapeDtypeStruct((S, D), jnp.float32)],
        scratch_shapes=[pltpu.VMEM((S * D, D), jnp.float32),
                        pltpu.VMEM((S, D), jnp.float32)],
        interpret=_INTERPRET,
    )(x, oh, cnts, wk, wv)


def _apply_body(s_act, final, *refs):
    if final:
        (x_ref, oh_ref, kv_ref, ks_ref, cnts_ref, wq_ref, wm_ref, m1_ref,
         m2_ref, ln_ref, orig_ref, gates_ref, y_ref) = refs
    else:
        (x_ref, oh_ref, kv_ref, ks_ref, cnts_ref, wq_ref, wm_ref, m1_ref,
         m2_ref, ln_ref, y_ref) = refs
    x = x_ref[...]
    oh = oh_ref[...]
    qq = _elu1(_dotbf(x, wq_ref[...], ((1,), (1,))))
    o = jnp.zeros((B, D), jnp.float32)
    for s in range(s_act):
        o = o + oh[:, s:s + 1] * _dotbf(qq, kv_ref[s * D:(s + 1) * D, :],
                                        ((1,), (0,)))
    ksel = _dothi(oh, ks_ref[...], ((1,), (0,)))
    ehr = (jax.lax.broadcasted_iota(jnp.int32, (D, H), 0) // DH ==
           jax.lax.broadcasted_iota(jnp.int32, (D, H), 1)).astype(jnp.float32)
    p = (qq.astype(jnp.bfloat16).astype(jnp.float32) *
         ksel.astype(jnp.bfloat16).astype(jnp.float32))
    den = _dothi(p, ehr, ((1,), (0,))) + 1e-6
    zf = _dothi(1.0 / den, ehr, ((1,), (1,)))
    cntsel = _dothi(oh, cnts_ref[:, 0:1], ((1,), (0,)))
    msg0 = (o * zf) * cntsel
    msg = _dotbf(msg0, wm_ref[...], ((1,), (1,)))
    mu = jnp.mean(msg, axis=1, keepdims=True)
    var = jnp.mean((msg - mu) ** 2, axis=1, keepdims=True)
    msg = (msg - mu) / jnp.sqrt(var + 1e-5) * ln_ref[0:1, :] + ln_ref[1:2, :]
    h = _dotbf(jnp.concatenate([x, msg], axis=1), m1_ref[...], ((1,), (1,)))
    h = jnp.maximum(h, 0.0)
    h2 = _dotbf(h, m2_ref[...], ((1,), (1,)))
    mu2 = jnp.mean(h2, axis=1, keepdims=True)
    var2 = jnp.mean((h2 - mu2) ** 2, axis=1, keepdims=True)
    msg2 = (h2 - mu2) / jnp.sqrt(var2 + 1e-5) * ln_ref[2:3, :] + ln_ref[3:4, :]
    y = x + msg2
    if final:
        g = _dothi(oh, gates_ref[:, 0:1], ((1,), (0,)))
        y = g * y + (1.0 - g) * orig_ref[...]
    y_ref[...] = y


def _apply(x, oh, kv, ks, cnts, wq, wm, m1, m2, ln, s_act,
           orig=None, gates=None):
    final = orig is not None
    body = lambda *refs: _apply_body(s_act, final, *refs)
    in_specs = [pl.BlockSpec((B, D), lambda b: (b, 0)),
                pl.BlockSpec((B, S), lambda b: (b, 0)),
                pl.BlockSpec((S * D, D), lambda b: (0, 0)),
                pl.BlockSpec((S, D), lambda b: (0, 0)),
                pl.BlockSpec((S, 128), lambda b: (0, 0)),
                pl.BlockSpec((D, D), lambda b: (0, 0)),
                pl.BlockSpec((D, D), lambda b: (0, 0)),
                pl.BlockSpec((2 * D, 2 * D), lambda b: (0, 0)),
                pl.BlockSpec((D, 2 * D), lambda b: (0, 0)),
                pl.BlockSpec((8, D), lambda b: (0, 0))]
    args = [x, oh, kv, ks, cnts, wq, wm, m1, m2, ln]
    if final:
        in_specs += [pl.BlockSpec((B, D), lambda b: (b, 0)),
                     pl.BlockSpec((S, 128), lambda b: (0, 0))]
        args += [orig, gates]
    return pl.pallas_call(
        body,
        grid=(NB,),
        in_specs=in_specs,
        out_specs=pl.BlockSpec((B, D), lambda b: (b, 0)),
        out_shape=jax.ShapeDtypeStruct((N, D), jnp.float32),
        interpret=_INTERPRET,
    )(*args)


def kernel(desc2d, desc3d,
           l1_q, l1_k, l1_v, l1_merge, l1_mlp1, l1_mlp2,
           l1_n1g, l1_n1b, l1_n2g, l1_n2b,
           l2_q, l2_k, l2_v, l2_merge, l2_mlp1, l2_mlp2,
           l2_n1g, l2_n1b, l2_n2g, l2_n2b):
    n2 = desc2d.shape[0]
    full = jnp.concatenate([desc2d, desc3d], axis=0)
    ln1 = jnp.concatenate([l1_n1g.reshape(1, D), l1_n1b.reshape(1, D),
                           l1_n2g.reshape(1, D), l1_n2b.reshape(1, D),
                           jnp.zeros((4, D), jnp.float32)], axis=0)
    ln2 = jnp.concatenate([l2_n1g.reshape(1, D), l2_n1b.reshape(1, D),
                           l2_n2g.reshape(1, D), l2_n2b.reshape(1, D),
                           jnp.zeros((4, D), jnp.float32)], axis=0)
    x2 = jnp.sum(full * full, axis=1, keepdims=True)
    x2b = jnp.broadcast_to(x2, (N, S))
    c = full[0:4, :]
    for _ in range(KM_ITERS):
        c2 = jnp.sum(c * c, axis=1)
        caug = jnp.zeros((S, D), jnp.float32).at[0:4, :].set(c)
        caug = caug.at[4, 0:4].set(c2)
        oh4 = _assign4(full, x2b, caug)
        cl = jnp.argmax(oh4[:, 0:4], axis=1).astype(jnp.int32)
        sums = jax.ops.segment_sum(full, cl, num_segments=4)
        counts = jnp.maximum(jnp.bincount(cl, length=4), 1).astype(full.dtype)
        c = sums / counts[:, None]
    cnt4 = jnp.bincount(cl, length=4)
    cnts4 = jnp.broadcast_to(
        jnp.pad(cnt4.astype(jnp.float32), (0, 4))[:, None], (S, 128))
    gates = jnp.broadcast_to(
        jnp.repeat((cnt4 > 20).astype(jnp.float32), 2)[:, None], (S, 128))

    kv1, ks1 = _segkv(full, oh4, cnts4, l1_k, l1_v, s_act=4)
    d0 = _apply(full, oh4, kv1, ks1, cnts4, l1_q, l1_merge, l1_mlp1, l1_mlp2,
                ln1, s_act=4)

    d0x2 = jnp.sum(d0 * d0, axis=1, keepdims=True)
    d0x2b = jnp.broadcast_to(d0x2, (N, S))
    ar = jnp.arange(N)
    crows = []
    for i in range(4):
        mask = (cl == i)
        order = jnp.argsort(jnp.where(mask, ar, N))
        crows.append(d0[order[0:2], :])
    c8 = jnp.concatenate(crows, axis=0)
    for _ in range(KM_ITERS):
        c2 = jnp.sum(c8 * c8, axis=1)
        caug = jnp.zeros((S + 1, D), jnp.float32).at[0:8, :].set(c8)
        caug = caug.at[8, 0:8].set(c2)
        clj = _assign8(d0, d0x2b, caug)
        jown = jnp.take_along_axis(clj, (2 * cl + 1)[:, None],
                                   axis=1)[:, 0].astype(jnp.int32)
        seg = 2 * cl + jown
        ohseg = (seg[:, None] == jnp.arange(8)[None, :]).astype(d0.dtype)
        sums = jnp.einsum('ts,td->sd', ohseg, d0,
                          precision=jax.lax.Precision.HIGHEST)
        counts = jnp.maximum(jnp.sum(ohseg, axis=0), 1.0)
        c8 = sums / counts[:, None]
    oh4f = (cl[:, None] == jnp.arange(4)[None, :]).astype(jnp.float32)
    oh8 = jnp.repeat(oh4f, 2, axis=1) * clj
    jsel = jnp.take_along_axis(clj, (2 * cl + 1)[:, None],
                               axis=1)[:, 0].astype(jnp.int32)
    cnt8 = jnp.bincount(2 * cl + jsel, length=8)
    cnts8 = jnp.broadcast_to(cnt8.astype(jnp.float32)[:, None], (S, 128))

    kv2, ks2 = _segkv(d0, oh8, cnts8, l2_k, l2_v, s_act=8)
    out = _apply(d0, oh8, kv2, ks2, cnts8, l2_q, l2_merge, l2_mlp1, l2_mlp2,
                 ln2, s_act=8, orig=full, gates=gates)
    return out[:n2, :], out[n2:, :]
